# baseline (speedup 1.0000x reference)
import functools

import jax
import jax.numpy as jnp
import numpy as np
from jax import lax
from jax.experimental import pallas as pl
from jax.experimental.pallas import tpu as pltpu
from jax.experimental.pallas import tpu_sc as plsc

BASE_AMP = 0.08


def _anchor_starts_np(signal_len, num_segments, seg_length):
    max_start = max(signal_len - seg_length, 0)
    head = 0.1 * signal_len
    tail = max(0.0, 0.78 * signal_len)
    anchors = np.linspace(head, tail, num_segments)
    return np.clip(np.round(anchors), 0, max_start).astype(np.int64)


def _row_kernel(pi_ref, pq_ref, scale_ref, row_ref, *, starts, seg_length,
                signal_len):
    pi = jnp.tanh(pi_ref[0, :])
    pq = jnp.tanh(pq_ref[0, :])
    row_ref[...] = jnp.zeros((2, signal_len), dtype=jnp.float32)
    for k, s in enumerate(starts):
        sc = jax.nn.relu(scale_ref[k])
        row_ref[0, pl.ds(s, seg_length)] = sc * pi
        row_ref[1, pl.ds(s, seg_length)] = sc * pq


def _build_row(pattern_i, pattern_q, segment_scale, starts, signal_len):
    seg_length = pattern_i.shape[0]
    return pl.pallas_call(
        functools.partial(_row_kernel, starts=starts, seg_length=seg_length,
                          signal_len=signal_len),
        in_specs=[
            pl.BlockSpec((1, seg_length), lambda: (0, 0)),
            pl.BlockSpec((1, seg_length), lambda: (0, 0)),
            pl.BlockSpec(memory_space=pltpu.SMEM),
        ],
        out_specs=pl.BlockSpec((2, signal_len), lambda: (0, 0)),
        out_shape=jax.ShapeDtypeStruct((2, signal_len), jnp.float32),
    )(pattern_i[None, :], pattern_q[None, :], segment_scale)


def kernel(x, pattern_i, pattern_q, segment_scale):
    batch, ch, signal_len = x.shape
    seg_length = pattern_i.shape[0]
    num_segments = segment_scale.shape[0]
    starts = [int(s) for s in
              _anchor_starts_np(signal_len, num_segments, seg_length)]

    row = _build_row(pattern_i, pattern_q, segment_scale, starts, signal_len)

    info = plsc.get_sparse_core_info()
    NC, NS, L = info.num_cores, info.num_subcores, info.num_lanes
    NW = NC * NS
    per_w = batch // NW
    n_chunks = signal_len // L
    inv_n = 1.0 / (ch * signal_len)
    mesh = plsc.VectorSubcoreMesh(core_axis_name="c", subcore_axis_name="s")

    @functools.partial(
        pl.kernel, mesh=mesh,
        out_type=jax.ShapeDtypeStruct((batch, ch, signal_len), jnp.float32),
        scratch_types=[
            pltpu.VMEM((ch, signal_len), jnp.float32),
            pltpu.VMEM((ch, signal_len), jnp.float32),
        ],
    )
    def sc_k(x_hbm, row_hbm, out_hbm, row_v, buf_v):
        wid = lax.axis_index("s") * NC + lax.axis_index("c")
        base = wid * per_w
        pltpu.sync_copy(row_hbm, row_v)

        def one_sample(j, carry):
            pltpu.sync_copy(x_hbm.at[base + j], buf_v)

            def acc_body(i, acc):
                v0 = buf_v[0, pl.ds(i * L, L)]
                v1 = buf_v[1, pl.ds(i * L, L)]
                return acc + v0 * v0 + v1 * v1

            acc = lax.fori_loop(0, n_chunks, acc_body,
                                jnp.zeros((L,), jnp.float32))
            lanes = lax.iota(jnp.int32, L)
            dnums = lax.GatherDimensionNumbers(
                offset_dims=(), collapsed_slice_dims=(0,),
                start_index_map=(0,))
            for k in (1, 2, 4, 8):
                shuf = lax.gather(
                    acc, (lanes ^ k)[:, None], dnums, slice_sizes=(1,),
                    mode=lax.GatherScatterMode.PROMISE_IN_BOUNDS)
                acc = acc + shuf
            mv = acc * inv_n + 1e-12
            yi = jnp.full((L,), 0x5F3759DF, dtype=jnp.int32) - (
                lax.bitcast_convert_type(mv, jnp.int32) >> 1)
            y = lax.bitcast_convert_type(yi, jnp.float32)
            for _ in range(4):
                y = y * (1.5 - 0.5 * mv * y * y)
            amp = BASE_AMP * mv * y

            def add_body(i, carry2):
                sl = pl.ds(i * L, L)
                buf_v[0, sl] = buf_v[0, sl] + amp * row_v[0, sl]
                buf_v[1, sl] = buf_v[1, sl] + amp * row_v[1, sl]
                return carry2

            lax.fori_loop(0, n_chunks, add_body, 0)
            pltpu.sync_copy(buf_v, out_hbm.at[base + j])
            return carry

        lax.fori_loop(0, per_w, one_sample, 0)

    return sc_k(x, row)


# speedup vs baseline: 2.0210x; 2.0210x over previous
import functools

import jax
import jax.numpy as jnp
import numpy as np
from jax import lax
from jax.experimental import pallas as pl
from jax.experimental.pallas import tpu as pltpu
from jax.experimental.pallas import tpu_sc as plsc

BASE_AMP = 0.08


def _anchor_starts_np(signal_len, num_segments, seg_length):
    max_start = max(signal_len - seg_length, 0)
    head = 0.1 * signal_len
    tail = max(0.0, 0.78 * signal_len)
    anchors = np.linspace(head, tail, num_segments)
    return np.clip(np.round(anchors), 0, max_start).astype(np.int64)


def _row_kernel(pi_ref, pq_ref, scale_ref, row_ref, *, starts, seg_length,
                signal_len):
    pi = jnp.tanh(pi_ref[0, :])
    pq = jnp.tanh(pq_ref[0, :])
    row_ref[...] = jnp.zeros((2, signal_len), dtype=jnp.float32)
    for k, s in enumerate(starts):
        sc = jax.nn.relu(scale_ref[k])
        row_ref[0, pl.ds(s, seg_length)] = sc * pi
        row_ref[1, pl.ds(s, seg_length)] = sc * pq


def _build_row(pattern_i, pattern_q, segment_scale, starts, signal_len):
    seg_length = pattern_i.shape[0]
    return pl.pallas_call(
        functools.partial(_row_kernel, starts=starts, seg_length=seg_length,
                          signal_len=signal_len),
        in_specs=[
            pl.BlockSpec((1, seg_length), lambda: (0, 0)),
            pl.BlockSpec((1, seg_length), lambda: (0, 0)),
            pl.BlockSpec(memory_space=pltpu.SMEM),
        ],
        out_specs=pl.BlockSpec((2, signal_len), lambda: (0, 0)),
        out_shape=jax.ShapeDtypeStruct((2, signal_len), jnp.float32),
    )(pattern_i[None, :], pattern_q[None, :], segment_scale)


def _tc_kernel(x_ref, row_ref, out_ref, *, inv_n):
    x0 = x_ref[:, 0, :]
    x1 = x_ref[:, 1, :]
    a = x0 * x0 + x1 * x1
    ss = jnp.sum(a, axis=1, keepdims=True)
    amp = BASE_AMP * jnp.sqrt(ss * inv_n + 1e-12)
    out_ref[...] = x_ref[...] + amp[:, :, None] * row_ref[...][None, :, :]


def _tc_part(x, row, n_tc, bb, inv_n):
    _, ch, signal_len = x.shape
    return pl.pallas_call(
        functools.partial(_tc_kernel, inv_n=inv_n),
        grid=(n_tc // bb,),
        in_specs=[
            pl.BlockSpec((bb, ch, signal_len), lambda i: (i, 0, 0)),
            pl.BlockSpec((ch, signal_len), lambda i: (0, 0)),
        ],
        out_specs=pl.BlockSpec((bb, ch, signal_len), lambda i: (i, 0, 0)),
        out_shape=jax.ShapeDtypeStruct((n_tc, ch, signal_len), jnp.float32),
    )(x, row)


def _sc_part(x, row, n_tc, n_sc, inv_n):
    batch, ch, signal_len = x.shape
    info = plsc.get_sparse_core_info()
    NC, NS, L = info.num_cores, info.num_subcores, info.num_lanes
    NW = NC * NS
    per_w = n_sc // NW
    n_chunks = signal_len // L
    mesh = plsc.VectorSubcoreMesh(core_axis_name="c", subcore_axis_name="s")

    @functools.partial(
        pl.kernel, mesh=mesh,
        out_type=jax.ShapeDtypeStruct((n_sc, ch, signal_len), jnp.float32),
        scratch_types=[
            pltpu.VMEM((ch, signal_len), jnp.float32),
            pltpu.VMEM((ch, signal_len), jnp.float32),
        ],
    )
    def sc_k(x_hbm, row_hbm, out_hbm, row_v, buf_v):
        wid = lax.axis_index("s") * NC + lax.axis_index("c")
        base = wid * per_w
        pltpu.sync_copy(row_hbm, row_v)

        def one_sample(j, carry):
            pltpu.sync_copy(x_hbm.at[n_tc + base + j], buf_v)

            def acc_body(i, acc):
                v0 = buf_v[0, pl.ds(i * L, L)]
                v1 = buf_v[1, pl.ds(i * L, L)]
                return acc + v0 * v0 + v1 * v1

            acc = lax.fori_loop(0, n_chunks, acc_body,
                                jnp.zeros((L,), jnp.float32))
            lanes = lax.iota(jnp.int32, L)
            dnums = lax.GatherDimensionNumbers(
                offset_dims=(), collapsed_slice_dims=(0,),
                start_index_map=(0,))
            for k in (1, 2, 4, 8):
                shuf = lax.gather(
                    acc, (lanes ^ k)[:, None], dnums, slice_sizes=(1,),
                    mode=lax.GatherScatterMode.PROMISE_IN_BOUNDS)
                acc = acc + shuf
            mv = acc * inv_n + 1e-12
            yi = jnp.full((L,), 0x5F3759DF, dtype=jnp.int32) - (
                lax.bitcast_convert_type(mv, jnp.int32) >> 1)
            y = lax.bitcast_convert_type(yi, jnp.float32)
            for _ in range(4):
                y = y * (1.5 - 0.5 * mv * y * y)
            amp = BASE_AMP * mv * y

            def add_body(i, carry2):
                sl = pl.ds(i * L, L)
                buf_v[0, sl] = buf_v[0, sl] + amp * row_v[0, sl]
                buf_v[1, sl] = buf_v[1, sl] + amp * row_v[1, sl]
                return carry2

            lax.fori_loop(0, n_chunks, add_body, 0)
            pltpu.sync_copy(buf_v, out_hbm.at[base + j])
            return carry

        lax.fori_loop(0, per_w, one_sample, 0)

    return sc_k(x, row)


def kernel(x, pattern_i, pattern_q, segment_scale):
    batch, ch, signal_len = x.shape
    seg_length = pattern_i.shape[0]
    num_segments = segment_scale.shape[0]
    starts = [int(s) for s in
              _anchor_starts_np(signal_len, num_segments, seg_length)]
    inv_n = 1.0 / (ch * signal_len)

    row = _build_row(pattern_i, pattern_q, segment_scale, starts, signal_len)

    n_sc = 160
    n_tc = batch - n_sc
    tc_out = _tc_part(x, row, n_tc, 96, inv_n)
    sc_out = _sc_part(x, row, n_tc, n_sc, inv_n)
    return jnp.concatenate([tc_out, sc_out], axis=0)


# speedup vs baseline: 4.8682x; 2.4088x over previous
import functools

import jax
import jax.numpy as jnp
import numpy as np
from jax.experimental import pallas as pl
from jax.experimental.pallas import tpu as pltpu

BASE_AMP = 0.08


def _anchor_starts_np(signal_len, num_segments, seg_length):
    max_start = max(signal_len - seg_length, 0)
    head = 0.1 * signal_len
    tail = max(0.0, 0.78 * signal_len)
    anchors = np.linspace(head, tail, num_segments)
    return np.clip(np.round(anchors), 0, max_start).astype(np.int64)


def _fused_kernel(x_ref, pi_ref, pq_ref, scale_ref, out_ref, row_ref, *,
                  starts, seg_length, signal_len, inv_n):
    @pl.when(pl.program_id(0) == 0)
    def _build_row():
        pi = jnp.tanh(pi_ref[0, :])
        pq = jnp.tanh(pq_ref[0, :])
        row_ref[...] = jnp.zeros((2, signal_len), dtype=jnp.float32)
        for k, s in enumerate(starts):
            sc = jax.nn.relu(scale_ref[k])
            row_ref[0, pl.ds(s, seg_length)] = sc * pi
            row_ref[1, pl.ds(s, seg_length)] = sc * pq

    x0 = x_ref[:, 0, :]
    x1 = x_ref[:, 1, :]
    a = x0 * x0 + x1 * x1
    ss = jnp.sum(a, axis=1, keepdims=True)
    amp = BASE_AMP * jnp.sqrt(ss * inv_n + 1e-12)
    out_ref[...] = x_ref[...] + amp[:, :, None] * row_ref[...][None, :, :]


def kernel(x, pattern_i, pattern_q, segment_scale):
    batch, ch, signal_len = x.shape
    seg_length = pattern_i.shape[0]
    num_segments = segment_scale.shape[0]
    starts = [int(s) for s in
              _anchor_starts_np(signal_len, num_segments, seg_length)]

    bb = 128
    grid = (batch // bb,)

    body = functools.partial(
        _fused_kernel, starts=starts, seg_length=seg_length,
        signal_len=signal_len, inv_n=1.0 / (ch * signal_len))

    return pl.pallas_call(
        body,
        grid=grid,
        in_specs=[
            pl.BlockSpec((bb, ch, signal_len), lambda i: (i, 0, 0)),
            pl.BlockSpec((1, seg_length), lambda i: (0, 0)),
            pl.BlockSpec((1, seg_length), lambda i: (0, 0)),
            pl.BlockSpec(memory_space=pltpu.SMEM),
        ],
        out_specs=pl.BlockSpec((bb, ch, signal_len), lambda i: (i, 0, 0)),
        out_shape=jax.ShapeDtypeStruct((batch, ch, signal_len), jnp.float32),
        scratch_shapes=[pltpu.VMEM((ch, signal_len), jnp.float32)],
    )(x, pattern_i[None, :], pattern_q[None, :], segment_scale)
